# Initial kernel scaffold; baseline (speedup 1.0000x reference)
import jax
import jax.numpy as jnp
from jax.experimental import pallas as pl


def kernel(x, edge_index, edge_weight, batch, W_emb, b_emb, W_conv, b_conv, Wc1, bc1, Wc2, bc2, Wc3, bc3):
    raise NotImplementedError("write your pallas kernel here")


# speedup vs baseline: 4.1803x; 4.1803x over previous
import functools

import jax
import jax.numpy as jnp
from jax import lax
from jax.experimental import pallas as pl
from jax.experimental.pallas import tpu as pltpu
from jax.experimental.pallas import tpu_sc as plsc

N_NODES = 10000
N_EDGES = 320000
F_IN = 128
HID = 256
HALF = 128
N_GRAPH = 64
N_CLS = 10

NC = 2
NS = 16
CH = 80
ROWS_A = 624
ZROWS = 208
TAIL = 16
TAIL_OFF = NS * ROWS_A

_mesh = plsc.VectorSubcoreMesh(core_axis_name="c", subcore_axis_name="s")


EPT_DEG = N_EDGES // (NC * NS)


@functools.partial(
    pl.kernel,
    mesh=_mesh,
    out_type=jax.ShapeDtypeStruct((NC * N_NODES, HALF), jnp.float32),
    scratch_types=[
        pltpu.VMEM((CH,), jnp.int32),
        pltpu.VMEM((CH,), jnp.float32),
        pltpu.VMEM((CH, HALF), jnp.float32),
        pltpu.VMEM((ZROWS, HALF), jnp.float32),
        pltpu.VMEM_SHARED((N_NODES, HALF), jnp.float32),
    ],
)
def _deg_call(idx_hbm, w_hbm, dp_hbm, idx_v, w_v, vals_v, zbuf, acc):
    c = lax.axis_index("c")
    s = lax.axis_index("s")
    wid = c * NS + s

    def zfill(i, _):
        for f in range(HALF // 16):
            zbuf[i, pl.ds(f * 16, 16)] = jnp.zeros((16,), jnp.float32)
        return 0

    lax.fori_loop(0, ZROWS, zfill, 0)
    for k in range(ROWS_A // ZROWS):
        pltpu.sync_copy(zbuf, acc.at[pl.ds(s * ROWS_A + k * ZROWS, ZROWS)])

    @pl.when(s == NS - 1)
    def _ztail():
        pltpu.sync_copy(zbuf.at[pl.ds(0, TAIL)], acc.at[pl.ds(TAIL_OFF, TAIL)])

    plsc.subcore_barrier()

    def chunk(i, _):
        base = wid * EPT_DEG + i * CH
        pltpu.sync_copy(idx_hbm.at[pl.ds(base, CH)], idx_v)
        pltpu.sync_copy(w_hbm.at[pl.ds(base, CH)], w_v)

        def fill(g, _):
            w16 = w_v[pl.ds(g * 16, 16)]
            for l in range(16):
                vals_v[g * 16 + l, pl.ds(0, 16)] = jnp.full((16,), w16[l],
                                                            jnp.float32)
            return 0

        lax.fori_loop(0, CH // 16, fill, 0)
        pltpu.sync_copy(vals_v, acc.at[idx_v], add=True)
        return 0

    lax.fori_loop(0, EPT_DEG // CH, chunk, 0)
    plsc.subcore_barrier()

    off = c * N_NODES + s * ROWS_A
    pltpu.sync_copy(acc.at[pl.ds(s * ROWS_A, ROWS_A)], dp_hbm.at[pl.ds(off, ROWS_A)])

    @pl.when(s == NS - 1)
    def _otail():
        pltpu.sync_copy(acc.at[pl.ds(TAIL_OFF, TAIL)],
                        dp_hbm.at[pl.ds(c * N_NODES + TAIL_OFF, TAIL)])


EPT_AGG = N_EDGES // NS


@functools.partial(
    pl.kernel,
    mesh=_mesh,
    out_type=jax.ShapeDtypeStruct((NC * N_NODES, HALF), jnp.float32),
    scratch_types=[
        pltpu.VMEM((CH,), jnp.int32),
        pltpu.VMEM((CH,), jnp.int32),
        pltpu.VMEM((CH,), jnp.float32),
        pltpu.VMEM((CH,), jnp.int32),
        pltpu.VMEM((CH, HALF), jnp.float32),
        pltpu.VMEM((ZROWS, HALF), jnp.float32),
        pltpu.VMEM_SHARED((N_NODES, HALF), jnp.float32),
        pltpu.SemaphoreType.DMA,
    ],
)
def _agg_call(hw_hbm, src_hbm, dst_hbm, w_hbm, out_hbm,
              src_v, dst_v, w_v, gidx_v, rows_v, zbuf, acc, sem):
    c = lax.axis_index("c")
    s = lax.axis_index("s")

    def zfill(i, _):
        for f in range(HALF // 16):
            zbuf[i, pl.ds(f * 16, 16)] = jnp.zeros((16,), jnp.float32)
        return 0

    lax.fori_loop(0, ZROWS, zfill, 0)
    for k in range(ROWS_A // ZROWS):
        pltpu.sync_copy(zbuf, acc.at[pl.ds(s * ROWS_A + k * ZROWS, ZROWS)])

    @pl.when(s == NS - 1)
    def _ztail():
        pltpu.sync_copy(zbuf.at[pl.ds(0, TAIL)], acc.at[pl.ds(TAIL_OFF, TAIL)])

    plsc.subcore_barrier()

    def chunk(i, _):
        base = s * EPT_AGG + i * CH
        pltpu.sync_copy(src_hbm.at[pl.ds(base, CH)], src_v)
        pltpu.sync_copy(dst_hbm.at[pl.ds(base, CH)], dst_v)
        pltpu.sync_copy(w_hbm.at[pl.ds(base, CH)], w_v)

        def addoff(k, _):
            gidx_v[pl.ds(k * 16, 16)] = src_v[pl.ds(k * 16, 16)] + c * N_NODES
            return 0

        lax.fori_loop(0, CH // 16, addoff, 0)
        pltpu.async_copy(hw_hbm.at[gidx_v], rows_v, sem).wait()

        def scale(g, _):
            w16 = w_v[pl.ds(g * 16, 16)]
            for l in range(16):
                wj = jnp.full((16,), w16[l], jnp.float32)
                j = g * 16 + l
                for f in range(HALF // 16):
                    rows_v[j, pl.ds(f * 16, 16)] = rows_v[j, pl.ds(f * 16, 16)] * wj
            return 0

        lax.fori_loop(0, CH // 16, scale, 0)
        pltpu.sync_copy(rows_v, acc.at[dst_v], add=True)
        return 0

    lax.fori_loop(0, EPT_AGG // CH, chunk, 0)
    plsc.subcore_barrier()

    off = c * N_NODES + s * ROWS_A
    pltpu.sync_copy(acc.at[pl.ds(s * ROWS_A, ROWS_A)], out_hbm.at[pl.ds(off, ROWS_A)])

    @pl.when(s == NS - 1)
    def _otail():
        pltpu.sync_copy(acc.at[pl.ds(TAIL_OFF, TAIL)],
                        out_hbm.at[pl.ds(c * N_NODES + TAIL_OFF, TAIL)])


def _tc_prep_body(x_ref, wemb_ref, bemb_ref, wconv_ref, dop_ref, dip_ref,
                  hws_ref, nsrc_ref, ndst_ref):
    deg_o = dop_ref[0:N_NODES, 0:1] + dop_ref[N_NODES:2 * N_NODES, 0:1]
    deg_i = dip_ref[0:N_NODES, 0:1] + dip_ref[N_NODES:2 * N_NODES, 0:1]
    nsrc = lax.rsqrt(jnp.maximum(deg_o, 1e-6))
    ndst = lax.rsqrt(jnp.maximum(deg_i, 1e-6))
    nsrc_ref[...] = nsrc
    ndst_ref[...] = ndst
    h0 = jnp.dot(x_ref[...], wemb_ref[...], preferred_element_type=jnp.float32)
    h0 = h0 + bemb_ref[...]
    hw = jnp.dot(h0, wconv_ref[...], preferred_element_type=jnp.float32) * nsrc
    hws_ref[0:N_NODES, :] = hw[:, 0:HALF]
    hws_ref[N_NODES:2 * N_NODES, :] = hw[:, HALF:HID]


_tc_prep = pl.pallas_call(
    _tc_prep_body,
    out_shape=[
        jax.ShapeDtypeStruct((NC * N_NODES, HALF), jnp.float32),
        jax.ShapeDtypeStruct((N_NODES, 1), jnp.float32),
        jax.ShapeDtypeStruct((N_NODES, 1), jnp.float32),
    ],
)


def _tc_layer_body(agg_ref, ndst_ref, nsrc_ref, bconv_ref, wconv_ref, hws_ref):
    h = jnp.concatenate(
        [agg_ref[0:N_NODES, :], agg_ref[N_NODES:2 * N_NODES, :]], axis=1)
    h = jnp.maximum(h * ndst_ref[...] + bconv_ref[...], 0.0)
    hw = jnp.dot(h, wconv_ref[...], preferred_element_type=jnp.float32)
    hw = hw * nsrc_ref[...]
    hws_ref[0:N_NODES, :] = hw[:, 0:HALF]
    hws_ref[N_NODES:2 * N_NODES, :] = hw[:, HALF:HID]


_tc_layer = pl.pallas_call(
    _tc_layer_body,
    out_shape=jax.ShapeDtypeStruct((NC * N_NODES, HALF), jnp.float32),
)


def _tc_head_body(agg_ref, ndst_ref, bconv_ref, onehot_ref,
                  wc1_ref, bc1_ref, wc2_ref, bc2_ref, wc3_ref, bc3_ref, out_ref):
    h = jnp.concatenate(
        [agg_ref[0:N_NODES, :], agg_ref[N_NODES:2 * N_NODES, :]], axis=1)
    h = h * ndst_ref[...] + bconv_ref[...]
    onehot = onehot_ref[...]
    sums = jnp.dot(onehot, h, preferred_element_type=jnp.float32)
    counts = jnp.sum(onehot, axis=1, keepdims=True)
    pooled = sums / jnp.maximum(counts, 1.0)
    o = jnp.maximum(jnp.dot(pooled, wc1_ref[...], preferred_element_type=jnp.float32) + bc1_ref[...], 0.0)
    o = jnp.maximum(jnp.dot(o, wc2_ref[...], preferred_element_type=jnp.float32) + bc2_ref[...], 0.0)
    out_ref[...] = jnp.dot(o, wc3_ref[...], preferred_element_type=jnp.float32) + bc3_ref[...]


_tc_head = pl.pallas_call(
    _tc_head_body,
    out_shape=jax.ShapeDtypeStruct((N_GRAPH, N_CLS), jnp.float32),
)


@jax.jit
def kernel(x, edge_index, edge_weight, batch, W_emb, b_emb, W_conv, b_conv,
           Wc1, bc1, Wc2, bc2, Wc3, bc3):
    src = edge_index[0]
    dst = edge_index[1]
    onehot = (batch[None, :] == jnp.arange(N_GRAPH, dtype=batch.dtype)[:, None]
              ).astype(jnp.float32)

    dop = _deg_call(src, edge_weight)
    dip = _deg_call(dst, edge_weight)
    hws, nsrc, ndst = _tc_prep(x, W_emb, b_emb.reshape(1, -1), W_conv[0],
                               dop, dip)
    agg = None
    for i in range(3):
        agg = _agg_call(hws, src, dst, edge_weight)
        if i < 2:
            hws = _tc_layer(agg, ndst, nsrc, b_conv[i].reshape(1, -1),
                            W_conv[i + 1])
    out = _tc_head(agg, ndst, b_conv[2].reshape(1, -1), onehot,
                   Wc1, bc1.reshape(1, -1), Wc2, bc2.reshape(1, -1),
                   Wc3, bc3.reshape(1, -1))
    return out


# speedup vs baseline: 9.0046x; 2.1541x over previous
import functools

import jax
import jax.numpy as jnp
from jax import lax
from jax.experimental import pallas as pl
from jax.experimental.pallas import tpu as pltpu
from jax.experimental.pallas import tpu_sc as plsc

N_NODES = 10000
N_EDGES = 320000
F_IN = 128
HID = 256
HALF = 128
N_GRAPH = 64
N_CLS = 10

NC = 2
NS = 16
CH = 80
ROWS_A = 624
ZROWS = 104
TAIL = 16
TAIL_OFF = NS * ROWS_A

_mesh = plsc.VectorSubcoreMesh(core_axis_name="c", subcore_axis_name="s")


EPT_DEG = N_EDGES // (NC * NS)


@functools.partial(
    pl.kernel,
    mesh=_mesh,
    out_type=jax.ShapeDtypeStruct((NC * N_NODES, HALF), jnp.float32),
    scratch_types=[
        pltpu.VMEM((CH,), jnp.int32),
        pltpu.VMEM((CH,), jnp.float32),
        pltpu.VMEM((CH, HALF), jnp.float32),
        pltpu.VMEM((ZROWS, HALF), jnp.float32),
        pltpu.VMEM_SHARED((N_NODES, HALF), jnp.float32),
    ],
)
def _deg_call(idx_hbm, w_hbm, dp_hbm, idx_v, w_v, vals_v, zbuf, acc):
    c = lax.axis_index("c")
    s = lax.axis_index("s")
    wid = c * NS + s

    def zfill(i, _):
        for f in range(HALF // 16):
            zbuf[i, pl.ds(f * 16, 16)] = jnp.zeros((16,), jnp.float32)
        return 0

    lax.fori_loop(0, ZROWS, zfill, 0)
    for k in range(ROWS_A // ZROWS):
        pltpu.sync_copy(zbuf, acc.at[pl.ds(s * ROWS_A + k * ZROWS, ZROWS)])

    @pl.when(s == NS - 1)
    def _ztail():
        pltpu.sync_copy(zbuf.at[pl.ds(0, TAIL)], acc.at[pl.ds(TAIL_OFF, TAIL)])

    plsc.subcore_barrier()

    def chunk(i, _):
        base = wid * EPT_DEG + i * CH
        pltpu.sync_copy(idx_hbm.at[pl.ds(base, CH)], idx_v)
        pltpu.sync_copy(w_hbm.at[pl.ds(base, CH)], w_v)

        def fill(g, _):
            w16 = w_v[pl.ds(g * 16, 16)]
            for l in range(16):
                vals_v[g * 16 + l, pl.ds(0, 16)] = jnp.full((16,), w16[l],
                                                            jnp.float32)
            return 0

        lax.fori_loop(0, CH // 16, fill, 0)
        pltpu.sync_copy(vals_v, acc.at[idx_v], add=True)
        return 0

    lax.fori_loop(0, EPT_DEG // CH, chunk, 0)
    plsc.subcore_barrier()

    off = c * N_NODES + s * ROWS_A
    pltpu.sync_copy(acc.at[pl.ds(s * ROWS_A, ROWS_A)], dp_hbm.at[pl.ds(off, ROWS_A)])

    @pl.when(s == NS - 1)
    def _otail():
        pltpu.sync_copy(acc.at[pl.ds(TAIL_OFF, TAIL)],
                        dp_hbm.at[pl.ds(c * N_NODES + TAIL_OFF, TAIL)])


EPT_AGG = N_EDGES // NS
NCH = EPT_AGG // CH
NP = NCH // 2


@functools.partial(
    pl.kernel,
    mesh=_mesh,
    out_type=jax.ShapeDtypeStruct((NC * N_NODES, HALF), jnp.float32),
    scratch_types=[
        pltpu.VMEM((CH,), jnp.int32),
        pltpu.VMEM((CH,), jnp.int32),
        pltpu.VMEM((CH,), jnp.int32),
        pltpu.VMEM((CH,), jnp.int32),
        pltpu.VMEM((CH,), jnp.float32),
        pltpu.VMEM((CH,), jnp.float32),
        pltpu.VMEM((CH, HALF), jnp.float32),
        pltpu.VMEM((CH, HALF), jnp.float32),
        pltpu.VMEM((ZROWS, HALF), jnp.float32),
        pltpu.VMEM_SHARED((N_NODES, HALF), jnp.float32),
        pltpu.SemaphoreType.DMA,
        pltpu.SemaphoreType.DMA,
        pltpu.SemaphoreType.DMA,
        pltpu.SemaphoreType.DMA,
        pltpu.SemaphoreType.DMA,
        pltpu.SemaphoreType.DMA,
        pltpu.SemaphoreType.DMA,
        pltpu.SemaphoreType.DMA,
    ],
)
def _agg_call(hw_hbm, src_hbm, dst_hbm, w_hbm, out_hbm,
              srcb0, srcb1, dstb0, dstb1, wb0, wb1,
              rows0, rows1, zbuf, acc,
              semG0, semG1, semS0, semS1, semLs0, semLs1, semLd0, semLd1):
    c = lax.axis_index("c")
    s = lax.axis_index("s")
    srcb = (srcb0, srcb1)
    dstb = (dstb0, dstb1)
    wb = (wb0, wb1)
    rows = (rows0, rows1)
    semG = (semG0, semG1)
    semS = (semS0, semS1)
    semLs = (semLs0, semLs1)
    semLd = (semLd0, semLd1)
    ebase = s * EPT_AGG
    coff = c * N_NODES

    def zfill(i, _):
        for f in range(HALF // 16):
            zbuf[i, pl.ds(f * 16, 16)] = jnp.zeros((16,), jnp.float32)
        return 0

    lax.fori_loop(0, ZROWS, zfill, 0)
    for k in range(ROWS_A // ZROWS):
        pltpu.sync_copy(zbuf, acc.at[pl.ds(s * ROWS_A + k * ZROWS, ZROWS)])

    @pl.when(s == NS - 1)
    def _ztail():
        pltpu.sync_copy(zbuf.at[pl.ds(0, TAIL)], acc.at[pl.ds(TAIL_OFF, TAIL)])

    plsc.subcore_barrier()

    def issue_ls(cc, b):
        pltpu.async_copy(src_hbm.at[pl.ds(ebase + cc * CH, CH)], srcb[b],
                         semLs[b])

    def issue_ld(cc, b):
        pltpu.async_copy(dst_hbm.at[pl.ds(ebase + cc * CH, CH)], dstb[b],
                         semLd[b])
        pltpu.async_copy(w_hbm.at[pl.ds(ebase + cc * CH, CH)], wb[b], semLd[b])

    def addoff(b):
        def body(k, _):
            srcb[b][pl.ds(k * 16, 16)] = srcb[b][pl.ds(k * 16, 16)] + coff
            return 0

        lax.fori_loop(0, CH // 16, body, 0)

    def issue_g(b):
        pltpu.async_copy(hw_hbm.at[srcb[b]], rows[b], semG[b])

    issue_ls(0, 0)
    issue_ls(1, 1)
    issue_ld(0, 0)
    pltpu.make_async_copy(src_hbm.at[pl.ds(ebase, CH)], srcb[0], semLs[0]).wait()
    addoff(0)
    issue_g(0)

    def when_opt(cond, fn):
        if cond is None:
            fn()
        else:
            pl.when(cond)(fn)

    def slot(cc, b, not_first, issue_next, issue_ls2):
        o = 1 - b
        pltpu.make_async_copy(hw_hbm.at[srcb[b]], rows[b], semG[b]).wait()

        when_opt(not_first, lambda: pltpu.make_async_copy(
            rows[o], acc.at[dstb[o]], semS[o]).wait())

        def start_next():
            pltpu.make_async_copy(
                src_hbm.at[pl.ds(ebase + (cc + 1) * CH, CH)], srcb[o],
                semLs[o]).wait()
            addoff(o)
            issue_g(o)
            issue_ld(cc + 1, o)

        when_opt(issue_next, start_next)
        when_opt(issue_ls2, lambda: issue_ls(cc + 2, b))

        pltpu.make_async_copy(
            dst_hbm.at[pl.ds(ebase + cc * CH, CH)], dstb[b], semLd[b]).wait()
        pltpu.make_async_copy(
            w_hbm.at[pl.ds(ebase + cc * CH, CH)], wb[b], semLd[b]).wait()

        def scale(g, _):
            w16 = wb[b][pl.ds(g * 16, 16)]
            for l in range(16):
                wj = jnp.full((16,), w16[l], jnp.float32)
                j = g * 16 + l
                for f in range(HALF // 16):
                    rows[b][j, pl.ds(f * 16, 16)] = rows[b][j, pl.ds(f * 16, 16)] * wj
            return 0

        lax.fori_loop(0, CH // 16, scale, 0)
        pltpu.async_copy(rows[b], acc.at[dstb[b]], semS[b], add=True)

    def pair(g, _):
        c0 = 2 * g
        slot(c0, 0, g >= 1, None, g < NP - 1)
        slot(c0 + 1, 1, None, g < NP - 1, g < NP - 1)
        return 0

    lax.fori_loop(0, NP, pair, 0)
    pltpu.make_async_copy(rows[1], acc.at[dstb[1]], semS[1]).wait()
    plsc.subcore_barrier()

    off = c * N_NODES + s * ROWS_A
    pltpu.sync_copy(acc.at[pl.ds(s * ROWS_A, ROWS_A)], out_hbm.at[pl.ds(off, ROWS_A)])

    @pl.when(s == NS - 1)
    def _otail():
        pltpu.sync_copy(acc.at[pl.ds(TAIL_OFF, TAIL)],
                        out_hbm.at[pl.ds(c * N_NODES + TAIL_OFF, TAIL)])


def _tc_prep_body(x_ref, wemb_ref, bemb_ref, wconv_ref, dop_ref, dip_ref,
                  hws_ref, nsrc_ref, ndst_ref):
    deg_o = dop_ref[0:N_NODES, 0:1] + dop_ref[N_NODES:2 * N_NODES, 0:1]
    deg_i = dip_ref[0:N_NODES, 0:1] + dip_ref[N_NODES:2 * N_NODES, 0:1]
    nsrc = lax.rsqrt(jnp.maximum(deg_o, 1e-6))
    ndst = lax.rsqrt(jnp.maximum(deg_i, 1e-6))
    nsrc_ref[...] = nsrc
    ndst_ref[...] = ndst
    h0 = jnp.dot(x_ref[...], wemb_ref[...], preferred_element_type=jnp.float32)
    h0 = h0 + bemb_ref[...]
    hw = jnp.dot(h0, wconv_ref[...], preferred_element_type=jnp.float32) * nsrc
    hws_ref[0:N_NODES, :] = hw[:, 0:HALF]
    hws_ref[N_NODES:2 * N_NODES, :] = hw[:, HALF:HID]


_tc_prep = pl.pallas_call(
    _tc_prep_body,
    out_shape=[
        jax.ShapeDtypeStruct((NC * N_NODES, HALF), jnp.float32),
        jax.ShapeDtypeStruct((N_NODES, 1), jnp.float32),
        jax.ShapeDtypeStruct((N_NODES, 1), jnp.float32),
    ],
)


def _tc_layer_body(agg_ref, ndst_ref, nsrc_ref, bconv_ref, wconv_ref, hws_ref):
    h = jnp.concatenate(
        [agg_ref[0:N_NODES, :], agg_ref[N_NODES:2 * N_NODES, :]], axis=1)
    h = jnp.maximum(h * ndst_ref[...] + bconv_ref[...], 0.0)
    hw = jnp.dot(h, wconv_ref[...], preferred_element_type=jnp.float32)
    hw = hw * nsrc_ref[...]
    hws_ref[0:N_NODES, :] = hw[:, 0:HALF]
    hws_ref[N_NODES:2 * N_NODES, :] = hw[:, HALF:HID]


_tc_layer = pl.pallas_call(
    _tc_layer_body,
    out_shape=jax.ShapeDtypeStruct((NC * N_NODES, HALF), jnp.float32),
)


def _tc_head_body(agg_ref, ndst_ref, bconv_ref, onehot_ref,
                  wc1_ref, bc1_ref, wc2_ref, bc2_ref, wc3_ref, bc3_ref, out_ref):
    h = jnp.concatenate(
        [agg_ref[0:N_NODES, :], agg_ref[N_NODES:2 * N_NODES, :]], axis=1)
    h = h * ndst_ref[...] + bconv_ref[...]
    onehot = onehot_ref[...]
    sums = jnp.dot(onehot, h, preferred_element_type=jnp.float32)
    counts = jnp.sum(onehot, axis=1, keepdims=True)
    pooled = sums / jnp.maximum(counts, 1.0)
    o = jnp.maximum(jnp.dot(pooled, wc1_ref[...], preferred_element_type=jnp.float32) + bc1_ref[...], 0.0)
    o = jnp.maximum(jnp.dot(o, wc2_ref[...], preferred_element_type=jnp.float32) + bc2_ref[...], 0.0)
    out_ref[...] = jnp.dot(o, wc3_ref[...], preferred_element_type=jnp.float32) + bc3_ref[...]


_tc_head = pl.pallas_call(
    _tc_head_body,
    out_shape=jax.ShapeDtypeStruct((N_GRAPH, N_CLS), jnp.float32),
)


@jax.jit
def kernel(x, edge_index, edge_weight, batch, W_emb, b_emb, W_conv, b_conv,
           Wc1, bc1, Wc2, bc2, Wc3, bc3):
    src = edge_index[0]
    dst = edge_index[1]
    onehot = (batch[None, :] == jnp.arange(N_GRAPH, dtype=batch.dtype)[:, None]
              ).astype(jnp.float32)

    dop = _deg_call(src, edge_weight)
    dip = _deg_call(dst, edge_weight)
    hws, nsrc, ndst = _tc_prep(x, W_emb, b_emb.reshape(1, -1), W_conv[0],
                               dop, dip)
    agg = None
    for i in range(3):
        agg = _agg_call(hws, src, dst, edge_weight)
        if i < 2:
            hws = _tc_layer(agg, ndst, nsrc, b_conv[i].reshape(1, -1),
                            W_conv[i + 1])
    out = _tc_head(agg, ndst, b_conv[2].reshape(1, -1), onehot,
                   Wc1, bc1.reshape(1, -1), Wc2, bc2.reshape(1, -1),
                   Wc3, bc3.reshape(1, -1))
    return out


# speedup vs baseline: 10.9617x; 1.2173x over previous
import functools

import jax
import jax.numpy as jnp
from jax import lax
from jax.experimental import pallas as pl
from jax.experimental.pallas import tpu as pltpu
from jax.experimental.pallas import tpu_sc as plsc

N_NODES = 10000
N_EDGES = 320000
F_IN = 128
HID = 256
HALF = 128
N_GRAPH = 64
N_CLS = 10

NC = 2
NS = 16
CH = 80
ROWS_A = 624
ZROWS = 104
TAIL = 16
TAIL_OFF = NS * ROWS_A

_mesh = plsc.VectorSubcoreMesh(core_axis_name="c", subcore_axis_name="s")


EPT_DEG = N_EDGES // (NC * NS)
NCH_D = EPT_DEG // CH
NP_D = NCH_D // 2
ZROWS_D = 16


@functools.partial(
    pl.kernel,
    mesh=_mesh,
    out_type=jax.ShapeDtypeStruct((NC * N_NODES, HALF), jnp.float32),
    scratch_types=[
        pltpu.VMEM((CH,), jnp.int32),
        pltpu.VMEM((CH,), jnp.int32),
        pltpu.VMEM((CH,), jnp.int32),
        pltpu.VMEM((CH,), jnp.int32),
        pltpu.VMEM((CH,), jnp.float32),
        pltpu.VMEM((CH,), jnp.float32),
        pltpu.VMEM((CH, HALF), jnp.float32),
        pltpu.VMEM((CH, HALF), jnp.float32),
        pltpu.VMEM((CH, HALF), jnp.float32),
        pltpu.VMEM((CH, HALF), jnp.float32),
        pltpu.VMEM((ZROWS_D, HALF), jnp.float32),
        pltpu.VMEM_SHARED((N_NODES, HALF), jnp.float32),
        pltpu.SemaphoreType.DMA,
        pltpu.SemaphoreType.DMA,
        pltpu.SemaphoreType.DMA,
        pltpu.SemaphoreType.DMA,
    ],
)
def _deg_call(src_hbm, dst_hbm, w_hbm, dp_hbm,
              srcb0, srcb1, dstb0, dstb1, wb0, wb1,
              valsA0, valsA1, valsB0, valsB1, zbuf, acc,
              semL0, semL1, semS0, semS1):
    c = lax.axis_index("c")
    s = lax.axis_index("s")
    srcb = (srcb0, srcb1)
    dstb = (dstb0, dstb1)
    wb = (wb0, wb1)
    valsA = (valsA0, valsA1)
    valsB = (valsB0, valsB1)
    semL = (semL0, semL1)
    semS = (semS0, semS1)
    wid = c * NS + s
    ebase = wid * EPT_DEG

    def vzero(i, _):
        for f in range(HALF // 16):
            for buf in (valsA0, valsA1, valsB0, valsB1):
                buf[i, pl.ds(f * 16, 16)] = jnp.zeros((16,), jnp.float32)
        return 0

    lax.fori_loop(0, CH, vzero, 0)

    def zfill(i, _):
        for f in range(HALF // 16):
            zbuf[i, pl.ds(f * 16, 16)] = jnp.zeros((16,), jnp.float32)
        return 0

    lax.fori_loop(0, ZROWS_D, zfill, 0)
    for k in range(ROWS_A // ZROWS_D):
        pltpu.sync_copy(zbuf, acc.at[pl.ds(s * ROWS_A + k * ZROWS_D, ZROWS_D)])

    @pl.when(s == NS - 1)
    def _ztail():
        pltpu.sync_copy(zbuf, acc.at[pl.ds(TAIL_OFF, TAIL)])

    plsc.subcore_barrier()

    def issue_l(cc, b):
        pltpu.async_copy(src_hbm.at[pl.ds(ebase + cc * CH, CH)], srcb[b], semL[b])
        pltpu.async_copy(dst_hbm.at[pl.ds(ebase + cc * CH, CH)], dstb[b], semL[b])
        pltpu.async_copy(w_hbm.at[pl.ds(ebase + cc * CH, CH)], wb[b], semL[b])

    def wait_l(cc, b):
        pltpu.make_async_copy(
            src_hbm.at[pl.ds(ebase + cc * CH, CH)], srcb[b], semL[b]).wait()
        pltpu.make_async_copy(
            dst_hbm.at[pl.ds(ebase + cc * CH, CH)], dstb[b], semL[b]).wait()
        pltpu.make_async_copy(
            w_hbm.at[pl.ds(ebase + cc * CH, CH)], wb[b], semL[b]).wait()

    def wait_s(b):
        pltpu.make_async_copy(valsA[b], acc.at[srcb[b]], semS[b]).wait()
        pltpu.make_async_copy(valsB[b], acc.at[dstb[b]], semS[b]).wait()

    def when_opt(cond, fn):
        if cond is None:
            fn()
        elif cond is not False:
            pl.when(cond)(fn)

    issue_l(0, 0)

    def slot(cc, b, not_first, issue_next):
        o = 1 - b
        when_opt(not_first, lambda: wait_s(o))
        when_opt(issue_next, lambda: issue_l(cc + 1, o))
        wait_l(cc, b)

        def fill(g, _):
            w16 = wb[b][pl.ds(g * 16, 16)]
            for l in range(16):
                wl = jnp.full((16,), w16[l], jnp.float32)
                valsA[b][g * 16 + l, pl.ds(0, 16)] = wl
                valsB[b][g * 16 + l, pl.ds(16, 16)] = wl
            return 0

        lax.fori_loop(0, CH // 16, fill, 0)
        pltpu.async_copy(valsA[b], acc.at[srcb[b]], semS[b], add=True)
        pltpu.async_copy(valsB[b], acc.at[dstb[b]], semS[b], add=True)

    def pair(g, _):
        c0 = 2 * g
        slot(c0, 0, g >= 1, None)
        slot(c0 + 1, 1, None, None)
        return 0

    lax.fori_loop(0, NP_D, pair, 0)
    slot(NCH_D - 1, 0, None, False)
    wait_s(0)
    plsc.subcore_barrier()

    off = c * N_NODES + s * ROWS_A
    pltpu.sync_copy(acc.at[pl.ds(s * ROWS_A, ROWS_A)], dp_hbm.at[pl.ds(off, ROWS_A)])

    @pl.when(s == NS - 1)
    def _otail():
        pltpu.sync_copy(acc.at[pl.ds(TAIL_OFF, TAIL)],
                        dp_hbm.at[pl.ds(c * N_NODES + TAIL_OFF, TAIL)])


EPT_AGG = N_EDGES // NS
NCH = EPT_AGG // CH
NP = NCH // 2


@functools.partial(
    pl.kernel,
    mesh=_mesh,
    out_type=jax.ShapeDtypeStruct((NC * N_NODES, HALF), jnp.float32),
    scratch_types=[
        pltpu.VMEM((CH,), jnp.int32),
        pltpu.VMEM((CH,), jnp.int32),
        pltpu.VMEM((CH,), jnp.int32),
        pltpu.VMEM((CH,), jnp.int32),
        pltpu.VMEM((CH,), jnp.float32),
        pltpu.VMEM((CH,), jnp.float32),
        pltpu.VMEM((CH, HALF), jnp.float32),
        pltpu.VMEM((CH, HALF), jnp.float32),
        pltpu.VMEM((ZROWS, HALF), jnp.float32),
        pltpu.VMEM_SHARED((N_NODES, HALF), jnp.float32),
        pltpu.SemaphoreType.DMA,
        pltpu.SemaphoreType.DMA,
        pltpu.SemaphoreType.DMA,
        pltpu.SemaphoreType.DMA,
        pltpu.SemaphoreType.DMA,
        pltpu.SemaphoreType.DMA,
        pltpu.SemaphoreType.DMA,
        pltpu.SemaphoreType.DMA,
    ],
)
def _agg_call(hw_hbm, src_hbm, dst_hbm, w_hbm, out_hbm,
              srcb0, srcb1, dstb0, dstb1, wb0, wb1,
              rows0, rows1, zbuf, acc,
              semG0, semG1, semS0, semS1, semLs0, semLs1, semLd0, semLd1):
    c = lax.axis_index("c")
    s = lax.axis_index("s")
    srcb = (srcb0, srcb1)
    dstb = (dstb0, dstb1)
    wb = (wb0, wb1)
    rows = (rows0, rows1)
    semG = (semG0, semG1)
    semS = (semS0, semS1)
    semLs = (semLs0, semLs1)
    semLd = (semLd0, semLd1)
    ebase = s * EPT_AGG
    coff = c * N_NODES

    def zfill(i, _):
        for f in range(HALF // 16):
            zbuf[i, pl.ds(f * 16, 16)] = jnp.zeros((16,), jnp.float32)
        return 0

    lax.fori_loop(0, ZROWS, zfill, 0)
    for k in range(ROWS_A // ZROWS):
        pltpu.sync_copy(zbuf, acc.at[pl.ds(s * ROWS_A + k * ZROWS, ZROWS)])

    @pl.when(s == NS - 1)
    def _ztail():
        pltpu.sync_copy(zbuf.at[pl.ds(0, TAIL)], acc.at[pl.ds(TAIL_OFF, TAIL)])

    plsc.subcore_barrier()

    def issue_ls(cc, b):
        pltpu.async_copy(src_hbm.at[pl.ds(ebase + cc * CH, CH)], srcb[b],
                         semLs[b])

    def issue_ld(cc, b):
        pltpu.async_copy(dst_hbm.at[pl.ds(ebase + cc * CH, CH)], dstb[b],
                         semLd[b])
        pltpu.async_copy(w_hbm.at[pl.ds(ebase + cc * CH, CH)], wb[b], semLd[b])

    def addoff(b):
        def body(k, _):
            srcb[b][pl.ds(k * 16, 16)] = srcb[b][pl.ds(k * 16, 16)] + coff
            return 0

        lax.fori_loop(0, CH // 16, body, 0)

    def issue_g(b):
        pltpu.async_copy(hw_hbm.at[srcb[b]], rows[b], semG[b])

    issue_ls(0, 0)
    issue_ls(1, 1)
    issue_ld(0, 0)
    pltpu.make_async_copy(src_hbm.at[pl.ds(ebase, CH)], srcb[0], semLs[0]).wait()
    addoff(0)
    issue_g(0)

    def when_opt(cond, fn):
        if cond is None:
            fn()
        else:
            pl.when(cond)(fn)

    def slot(cc, b, not_first, issue_next, issue_ls2):
        o = 1 - b
        pltpu.make_async_copy(hw_hbm.at[srcb[b]], rows[b], semG[b]).wait()

        when_opt(not_first, lambda: pltpu.make_async_copy(
            rows[o], acc.at[dstb[o]], semS[o]).wait())

        def start_next():
            pltpu.make_async_copy(
                src_hbm.at[pl.ds(ebase + (cc + 1) * CH, CH)], srcb[o],
                semLs[o]).wait()
            addoff(o)
            issue_g(o)
            issue_ld(cc + 1, o)

        when_opt(issue_next, start_next)
        when_opt(issue_ls2, lambda: issue_ls(cc + 2, b))

        pltpu.make_async_copy(
            dst_hbm.at[pl.ds(ebase + cc * CH, CH)], dstb[b], semLd[b]).wait()
        pltpu.make_async_copy(
            w_hbm.at[pl.ds(ebase + cc * CH, CH)], wb[b], semLd[b]).wait()

        def scale(g, _):
            w16 = wb[b][pl.ds(g * 16, 16)]
            for l in range(16):
                wj = jnp.full((16,), w16[l], jnp.float32)
                j = g * 16 + l
                for f in range(HALF // 16):
                    rows[b][j, pl.ds(f * 16, 16)] = rows[b][j, pl.ds(f * 16, 16)] * wj
            return 0

        lax.fori_loop(0, CH // 16, scale, 0)
        pltpu.async_copy(rows[b], acc.at[dstb[b]], semS[b], add=True)

    def pair(g, _):
        c0 = 2 * g
        slot(c0, 0, g >= 1, None, g < NP - 1)
        slot(c0 + 1, 1, None, g < NP - 1, g < NP - 1)
        return 0

    lax.fori_loop(0, NP, pair, 0)
    pltpu.make_async_copy(rows[1], acc.at[dstb[1]], semS[1]).wait()
    plsc.subcore_barrier()

    off = c * N_NODES + s * ROWS_A
    pltpu.sync_copy(acc.at[pl.ds(s * ROWS_A, ROWS_A)], out_hbm.at[pl.ds(off, ROWS_A)])

    @pl.when(s == NS - 1)
    def _otail():
        pltpu.sync_copy(acc.at[pl.ds(TAIL_OFF, TAIL)],
                        out_hbm.at[pl.ds(c * N_NODES + TAIL_OFF, TAIL)])


def _tc_prep_body(x_ref, wemb_ref, bemb_ref, wconv_ref, dp_ref,
                  hws_ref, nsrc_ref, ndst_ref):
    deg_o = dp_ref[0:N_NODES, 0:1] + dp_ref[N_NODES:2 * N_NODES, 0:1]
    deg_i = dp_ref[0:N_NODES, 16:17] + dp_ref[N_NODES:2 * N_NODES, 16:17]
    nsrc = lax.rsqrt(jnp.maximum(deg_o, 1e-6))
    ndst = lax.rsqrt(jnp.maximum(deg_i, 1e-6))
    nsrc_ref[...] = nsrc
    ndst_ref[...] = ndst
    h0 = jnp.dot(x_ref[...], wemb_ref[...], preferred_element_type=jnp.float32)
    h0 = h0 + bemb_ref[...]
    hw = jnp.dot(h0, wconv_ref[...], preferred_element_type=jnp.float32) * nsrc
    hws_ref[0:N_NODES, :] = hw[:, 0:HALF]
    hws_ref[N_NODES:2 * N_NODES, :] = hw[:, HALF:HID]


_tc_prep = pl.pallas_call(
    _tc_prep_body,
    out_shape=[
        jax.ShapeDtypeStruct((NC * N_NODES, HALF), jnp.float32),
        jax.ShapeDtypeStruct((N_NODES, 1), jnp.float32),
        jax.ShapeDtypeStruct((N_NODES, 1), jnp.float32),
    ],
)


def _tc_layer_body(agg_ref, ndst_ref, nsrc_ref, bconv_ref, wconv_ref, hws_ref):
    h = jnp.concatenate(
        [agg_ref[0:N_NODES, :], agg_ref[N_NODES:2 * N_NODES, :]], axis=1)
    h = jnp.maximum(h * ndst_ref[...] + bconv_ref[...], 0.0)
    hw = jnp.dot(h, wconv_ref[...], preferred_element_type=jnp.float32)
    hw = hw * nsrc_ref[...]
    hws_ref[0:N_NODES, :] = hw[:, 0:HALF]
    hws_ref[N_NODES:2 * N_NODES, :] = hw[:, HALF:HID]


_tc_layer = pl.pallas_call(
    _tc_layer_body,
    out_shape=jax.ShapeDtypeStruct((NC * N_NODES, HALF), jnp.float32),
)


def _tc_head_body(agg_ref, ndst_ref, bconv_ref, onehot_ref,
                  wc1_ref, bc1_ref, wc2_ref, bc2_ref, wc3_ref, bc3_ref, out_ref):
    h = jnp.concatenate(
        [agg_ref[0:N_NODES, :], agg_ref[N_NODES:2 * N_NODES, :]], axis=1)
    h = h * ndst_ref[...] + bconv_ref[...]
    onehot = onehot_ref[...]
    sums = jnp.dot(onehot, h, preferred_element_type=jnp.float32)
    counts = jnp.sum(onehot, axis=1, keepdims=True)
    pooled = sums / jnp.maximum(counts, 1.0)
    o = jnp.maximum(jnp.dot(pooled, wc1_ref[...], preferred_element_type=jnp.float32) + bc1_ref[...], 0.0)
    o = jnp.maximum(jnp.dot(o, wc2_ref[...], preferred_element_type=jnp.float32) + bc2_ref[...], 0.0)
    out_ref[...] = jnp.dot(o, wc3_ref[...], preferred_element_type=jnp.float32) + bc3_ref[...]


_tc_head = pl.pallas_call(
    _tc_head_body,
    out_shape=jax.ShapeDtypeStruct((N_GRAPH, N_CLS), jnp.float32),
)


@jax.jit
def kernel(x, edge_index, edge_weight, batch, W_emb, b_emb, W_conv, b_conv,
           Wc1, bc1, Wc2, bc2, Wc3, bc3):
    src = edge_index[0]
    dst = edge_index[1]
    onehot = (batch[None, :] == jnp.arange(N_GRAPH, dtype=batch.dtype)[:, None]
              ).astype(jnp.float32)

    dp = _deg_call(src, dst, edge_weight)
    hws, nsrc, ndst = _tc_prep(x, W_emb, b_emb.reshape(1, -1), W_conv[0], dp)
    agg = None
    for i in range(3):
        agg = _agg_call(hws, src, dst, edge_weight)
        if i < 2:
            hws = _tc_layer(agg, ndst, nsrc, b_conv[i].reshape(1, -1),
                            W_conv[i + 1])
    out = _tc_head(agg, ndst, b_conv[2].reshape(1, -1), onehot,
                   Wc1, bc1.reshape(1, -1), Wc2, bc2.reshape(1, -1),
                   Wc3, bc3.reshape(1, -1))
    return out
